# baseline (speedup 1.0000x reference)
import jax
import jax.numpy as jnp
from jax import lax
from jax.experimental import pallas as pl
from jax.experimental.pallas import tpu as pltpu

_VOCAB = 100000
_EMBED = 64
_B = 16
_L = 16
_TOKENS = _B * _L
_VT = 2048
_NFULL = _VOCAB // _VT
_TAIL = _VOCAB - _NFULL * _VT
_NG = 8
_NW = 6
_NO = 12
_NSUP = _NFULL // _NO


def _body(x_sr, table_r, w_r, b_ref, out_r,
          emb_v, wbufs, obufs, wtail, otail, gsems, wsems, osems, tsems):
    def _g_dma(i):
        return pltpu.make_async_copy(
            table_r.at[pl.ds(x_sr[i], 1), :],
            emb_v.at[pl.ds(i, 1), :],
            gsems.at[lax.rem(i, _NG)])

    def _w_dma(j, slot):
        return pltpu.make_async_copy(
            w_r.at[pl.ds(j * _VT, _VT), :],
            wbufs.at[slot],
            wsems.at[slot])

    def _o_dma(j, slot):
        return pltpu.make_async_copy(
            obufs.at[slot],
            out_r.at[:, :, pl.ds(j * _VT, _VT)],
            osems.at[slot])

    for k in range(_NW):
        _w_dma(k, k).start()
    lax.fori_loop(0, _TOKENS, lambda i, c: (_g_dma(i).start(), c)[1], 0,
                  unroll=8)
    lax.fori_loop(0, _TOKENS, lambda i, c: (_g_dma(i).wait(), c)[1], 0,
                  unroll=8)
    emb = emb_v[...]

    def sub_step(j, k):
        _w_dma(j, k % _NW).wait()

        @pl.when(j >= _NO)
        def _():
            _o_dma(j - _NO, k).wait()

        acc = lax.dot_general(
            emb, wbufs[k % _NW],
            dimension_numbers=(((1,), (1,)), ((), ())),
            preferred_element_type=jnp.float32,
        ) + b_ref[0, pl.ds(j * _VT, _VT)]
        obufs[k] = acc.reshape(_B, _L, _VT)
        pltpu.async_copy(
            obufs.at[k],
            out_r.at[:, :, pl.ds(j * _VT, _VT)],
            osems.at[k], priority=1)

        @pl.when(j + _NW < _NFULL)
        def _():
            _w_dma(j + _NW, k % _NW).start()

    def super_step(s, c):
        for k in range(_NO):
            sub_step(s * _NO + k, k)
        return c

    lax.fori_loop(0, _NSUP, super_step, 0)

    wt_dma = pltpu.make_async_copy(
        w_r.at[pl.ds(_NFULL * _VT, _TAIL), :], wtail, tsems.at[0])
    ot_dma = pltpu.make_async_copy(
        otail, out_r.at[:, :, pl.ds(_NFULL * _VT, _TAIL)], tsems.at[1])
    wt_dma.start()
    wt_dma.wait()
    acc = lax.dot_general(
        emb, wtail[...],
        dimension_numbers=(((1,), (1,)), ((), ())),
        preferred_element_type=jnp.float32,
    ) + b_ref[0, pl.ds(_NFULL * _VT, _TAIL)]
    otail[...] = acc.reshape(_B, _L, _TAIL)
    ot_dma.start()

    for j in range(_NFULL - _NO, _NFULL):
        _o_dma(j, j % _NO).wait()
    ot_dma.wait()


def kernel(x, embed_table, W, b):
    x_flat = x.reshape(-1).astype(jnp.int32)

    out = pl.pallas_call(
        _body,
        in_specs=[
            pl.BlockSpec(memory_space=pltpu.SMEM),
            pl.BlockSpec(memory_space=pltpu.HBM),
            pl.BlockSpec(memory_space=pltpu.HBM),
            pl.BlockSpec((1, _VOCAB), lambda: (0, 0)),
        ],
        out_specs=pl.BlockSpec(memory_space=pltpu.HBM),
        out_shape=jax.ShapeDtypeStruct((_B, _L, _VOCAB), jnp.float32),
        compiler_params=pltpu.CompilerParams(
            vmem_limit_bytes=100 * 1024 * 1024),
        scratch_shapes=[
            pltpu.VMEM((_TOKENS, _EMBED), jnp.float32),
            pltpu.VMEM((_NW, _VT, _EMBED), jnp.float32),
            pltpu.VMEM((_NO, _B, _L, _VT), jnp.float32),
            pltpu.VMEM((_TAIL, _EMBED), jnp.float32),
            pltpu.VMEM((_B, _L, _TAIL), jnp.float32),
            pltpu.SemaphoreType.DMA((_NG,)),
            pltpu.SemaphoreType.DMA((_NW,)),
            pltpu.SemaphoreType.DMA((_NO,)),
            pltpu.SemaphoreType.DMA((2,)),
        ],
    )(x_flat, embed_table, W, b.reshape(1, _VOCAB))

    return out
